# baseline (device time: 192757 ns/iter reference)
import jax
import jax.numpy as jnp
from jax import lax
from jax.experimental import pallas as pl
from jax.experimental.pallas import tpu as pltpu

N_DEV = 4
T = 4096
D = 2048
HD = D // 2
V_SHARD = 8192
C = T // N_DEV

_GROUP_OFFSET = (0, -1, 1, 2)


def kernel(ids, E):
    mids = jnp.bitwise_and(ids, V_SHARD - 1)

    def body(mids_smem, ids_v, e_hbm, out_hbm, g_ref, rs_recv, accbuf,
             gsem, outsem, rs_send_sems, rs_recv_sems,
             ag_send_sems, ag_recv_sems):
        my = lax.axis_index("i")
        left = lax.rem(my + N_DEV - 1, N_DEV)
        right = lax.rem(my + 1, N_DEV)
        base = my * V_SHARD

        def group_chunk(o):
            return lax.rem(my + _GROUP_OFFSET[o] + N_DEV, N_DEV)

        def issue_group(o):
            t0 = group_chunk(o) * C

            def st(i, carry):
                l = mids_smem[t0 + i]
                pltpu.make_async_copy(
                    e_hbm.at[pl.ds(l, 1), :],
                    g_ref.at[pl.ds(t0 + i, 1), :],
                    gsem.at[o],
                ).start()
                return carry

            lax.fori_loop(0, C, st, 0, unroll=8)

        def drain_group(o):
            pltpu.make_async_copy(
                e_hbm.at[pl.ds(0, C), :],
                g_ref.at[pl.ds(group_chunk(o) * C, C), :],
                gsem.at[o],
            ).wait()

        def xval(c, d):
            idv = ids_v[pl.ds(c * C, C), :]
            ok = (idv >= base) & (idv < base + V_SHARD)
            gg = g_ref[pl.ds(c * C, C), pl.ds(d * HD, HD)]
            return jnp.where(ok, gg, 0.0).astype(jnp.bfloat16)

        dests = (right, left)

        def rchunk(d, s):
            step = (s + 1) if d else -(s + 1)
            return lax.rem(my + step + 2 * N_DEV, N_DEV)

        issue_group(0)
        drain_group(0)
        accbuf[0] = xval(my, 0)
        accbuf[1] = xval(my, 1)

        barrier_sem = pltpu.get_barrier_semaphore()
        for nbr in [left, right]:
            pl.semaphore_signal(
                barrier_sem, inc=1,
                device_id=(nbr,), device_id_type=pl.DeviceIdType.MESH,
            )
        pl.semaphore_wait(barrier_sem, 2)

        for s in range(N_DEV - 1):
            rdmas = []
            for d in (0, 1):
                rdma = pltpu.make_async_remote_copy(
                    src_ref=accbuf.at[d] if s == 0 else rs_recv.at[d, s - 1],
                    dst_ref=rs_recv.at[d, s],
                    send_sem=rs_send_sems.at[d, s],
                    recv_sem=rs_recv_sems.at[d, s],
                    device_id=(dests[d],),
                    device_id_type=pl.DeviceIdType.MESH,
                )
                rdma.start()
                rdmas.append(rdma)
            if s == 0:
                issue_group(1)
                issue_group(2)
                drain_group(1)
                drain_group(2)
            elif s == 1:
                issue_group(3)
                drain_group(3)
            for d, rdma in enumerate(rdmas):
                rdma.wait()
                rc = rchunk(d, s)
                acc = rs_recv[d, s] + xval(rc, d)
                if s < N_DEV - 2:
                    rs_recv[d, s] = acc
                else:
                    accbuf[d] = acc
                    pltpu.make_async_copy(
                        accbuf.at[d],
                        out_hbm.at[pl.ds(rc * C, C), pl.ds(d * HD, HD)],
                        outsem,
                    ).start()
        for d in (0, 1):
            pltpu.make_async_copy(
                accbuf.at[d],
                out_hbm.at[pl.ds(rchunk(d, N_DEV - 2) * C, C),
                           pl.ds(d * HD, HD)],
                outsem,
            ).wait()

        for h in range(N_DEV - 1):
            rdmas = []
            for d in (0, 1):
                gc = lax.rem(my + (h - 1 if d else 1 - h) + 2 * N_DEV, N_DEV)
                sl = out_hbm.at[pl.ds(gc * C, C), pl.ds(d * HD, HD)]
                rdma = pltpu.make_async_remote_copy(
                    src_ref=sl,
                    dst_ref=sl,
                    send_sem=ag_send_sems.at[d, h],
                    recv_sem=ag_recv_sems.at[d, h],
                    device_id=(dests[d],),
                    device_id_type=pl.DeviceIdType.MESH,
                )
                rdma.start()
                rdmas.append(rdma)
            for rdma in rdmas:
                rdma.wait()

    return pl.pallas_call(
        body,
        out_shape=jax.ShapeDtypeStruct((T, D), jnp.bfloat16),
        in_specs=[
            pl.BlockSpec(memory_space=pltpu.SMEM),
            pl.BlockSpec(memory_space=pltpu.VMEM),
            pl.BlockSpec(memory_space=pl.ANY),
        ],
        out_specs=pl.BlockSpec(memory_space=pl.ANY),
        scratch_shapes=[
            pltpu.VMEM((T, D), jnp.float32),
            pltpu.VMEM((2, N_DEV - 1, C, HD), jnp.bfloat16),
            pltpu.VMEM((2, C, HD), jnp.bfloat16),
            pltpu.SemaphoreType.DMA((N_DEV,)),
            pltpu.SemaphoreType.DMA,
            pltpu.SemaphoreType.DMA((2, N_DEV - 1)),
            pltpu.SemaphoreType.DMA((2, N_DEV - 1)),
            pltpu.SemaphoreType.DMA((2, N_DEV - 1)),
            pltpu.SemaphoreType.DMA((2, N_DEV - 1)),
        ],
        compiler_params=pltpu.CompilerParams(
            collective_id=0,
            vmem_limit_bytes=60 * 1024 * 1024,
        ),
    )(mids, ids.reshape(T, 1), E)


# device time: 182675 ns/iter; 1.0552x vs baseline; 1.0552x over previous
import jax
import jax.numpy as jnp
from jax import lax
from jax.experimental import pallas as pl
from jax.experimental.pallas import tpu as pltpu

N_DEV = 4
T = 4096
D = 2048
HD = D // 2
V_SHARD = 8192
C = T // N_DEV
SUB = 2
S = C // SUB

_GROUP_OFFSET = (0, -1, 1, 2)


def kernel(ids, E):
    mids = jnp.bitwise_and(ids, V_SHARD - 1)

    def body(mids_smem, ids_v, e_hbm, out_hbm, g_ref, rs_recv, accbuf,
             gsem, outsem, snd_sems, rs_recv_sems, ag_recv_sems):
        my = lax.axis_index("i")
        left = lax.rem(my + N_DEV - 1, N_DEV)
        right = lax.rem(my + 1, N_DEV)
        base = my * V_SHARD

        def group_chunk(o):
            return lax.rem(my + _GROUP_OFFSET[o] + N_DEV, N_DEV)

        def issue_sub(o, j):
            t0 = group_chunk(o) * C + j * S

            def st(i, carry):
                l = mids_smem[t0 + i]
                pltpu.make_async_copy(
                    e_hbm.at[pl.ds(l, 1), :],
                    g_ref.at[pl.ds(t0 + i, 1), :],
                    gsem.at[o, j],
                ).start()
                return carry

            lax.fori_loop(0, S, st, 0, unroll=8)

        def drain_sub(o, j):
            pltpu.make_async_copy(
                e_hbm.at[pl.ds(0, S), :],
                g_ref.at[pl.ds(group_chunk(o) * C, S), :],
                gsem.at[o, j],
            ).wait()

        def xval(c, d, j):
            idv = ids_v[pl.ds(c * C + j * S, S), :]
            ok = (idv >= base) & (idv < base + V_SHARD)
            gg = g_ref[pl.ds(c * C + j * S, S), pl.ds(d * HD, HD)]
            return jnp.where(ok, gg, 0.0).astype(jnp.bfloat16)

        dests = (right, left)

        def rchunk(d, s):
            step = (s + 1) if d else -(s + 1)
            return lax.rem(my + step + 2 * N_DEV, N_DEV)

        def rs_send(s, d, j):
            if s == 0:
                src = accbuf.at[d, pl.ds(j * S, S), :]
            else:
                src = rs_recv.at[d, s - 1, pl.ds(j * S, S), :]
            return pltpu.make_async_remote_copy(
                src_ref=src,
                dst_ref=rs_recv.at[d, s, pl.ds(j * S, S), :],
                send_sem=snd_sems.at[d, s, j],
                recv_sem=rs_recv_sems.at[d, s, j],
                device_id=(dests[d],),
                device_id_type=pl.DeviceIdType.MESH,
            )

        def ag_sl(h, d, j):
            gc = lax.rem(my + (h - 1 if d else 1 - h) + 2 * N_DEV, N_DEV)
            return out_hbm.at[pl.ds(gc * C + j * S, S), pl.ds(d * HD, HD)]

        def ag_send(h, d, j):
            sl = ag_sl(h, d, j)
            return pltpu.make_async_remote_copy(
                src_ref=sl,
                dst_ref=sl,
                send_sem=snd_sems.at[d, h, j],
                recv_sem=ag_recv_sems.at[d, h, j],
                device_id=(dests[d],),
                device_id_type=pl.DeviceIdType.MESH,
            )

        issue_sub(0, 0)
        issue_sub(0, 1)
        for j in range(SUB):
            drain_sub(0, j)
            for d in (0, 1):
                accbuf[d, pl.ds(j * S, S), :] = xval(my, d, j)

        barrier_sem = pltpu.get_barrier_semaphore()
        for nbr in [left, right]:
            pl.semaphore_signal(
                barrier_sem, inc=1,
                device_id=(nbr,), device_id_type=pl.DeviceIdType.MESH,
            )
        pl.semaphore_wait(barrier_sem, 2)

        rs_rdmas = []
        for d in (0, 1):
            for j in range(SUB):
                r = rs_send(0, d, j)
                r.start()
                rs_rdmas.append(r)
        issue_sub(1, 0)
        issue_sub(2, 0)
        issue_sub(1, 1)
        issue_sub(2, 1)
        for s in range(N_DEV - 1):
            if s == 1:
                issue_sub(3, 0)
                issue_sub(3, 1)
            for j in range(SUB):
                if s == 0:
                    drain_sub(1, j)
                    drain_sub(2, j)
                elif s == 1:
                    drain_sub(3, j)
                for d in (0, 1):
                    rs_send(s, d, j).wait_recv()
                    rc = rchunk(d, s)
                    acc = (rs_recv[d, s, pl.ds(j * S, S), :]
                           + xval(rc, d, j))
                    if s < N_DEV - 2:
                        rs_recv[d, s, pl.ds(j * S, S), :] = acc
                        r = rs_send(s + 1, d, j)
                        r.start()
                        rs_rdmas.append(r)
                    else:
                        accbuf[d, pl.ds(j * S, S), :] = acc
                        pltpu.make_async_copy(
                            accbuf.at[d, pl.ds(j * S, S), :],
                            out_hbm.at[pl.ds(rc * C + j * S, S),
                                       pl.ds(d * HD, HD)],
                            outsem.at[d, j],
                        ).start()
        for r in rs_rdmas:
            r.wait_send()

        ag_rdmas = []
        for j in range(SUB):
            for d in (0, 1):
                pltpu.make_async_copy(
                    accbuf.at[d, pl.ds(j * S, S), :],
                    out_hbm.at[pl.ds(rchunk(d, N_DEV - 2) * C + j * S, S),
                               pl.ds(d * HD, HD)],
                    outsem.at[d, j],
                ).wait()
                r = ag_send(0, d, j)
                r.start()
                ag_rdmas.append(r)
        for h in range(N_DEV - 1):
            for j in range(SUB):
                for d in (0, 1):
                    ag_send(h, d, j).wait_recv()
                    if h < N_DEV - 2:
                        r = ag_send(h + 1, d, j)
                        r.start()
                        ag_rdmas.append(r)
        for r in ag_rdmas:
            r.wait_send()

    return pl.pallas_call(
        body,
        out_shape=jax.ShapeDtypeStruct((T, D), jnp.bfloat16),
        in_specs=[
            pl.BlockSpec(memory_space=pltpu.SMEM),
            pl.BlockSpec(memory_space=pltpu.VMEM),
            pl.BlockSpec(memory_space=pl.ANY),
        ],
        out_specs=pl.BlockSpec(memory_space=pl.ANY),
        scratch_shapes=[
            pltpu.VMEM((T, D), jnp.float32),
            pltpu.VMEM((2, N_DEV - 1, C, HD), jnp.bfloat16),
            pltpu.VMEM((2, C, HD), jnp.bfloat16),
            pltpu.SemaphoreType.DMA((N_DEV, SUB)),
            pltpu.SemaphoreType.DMA((2, SUB)),
            pltpu.SemaphoreType.DMA((2, N_DEV - 1, SUB)),
            pltpu.SemaphoreType.DMA((2, N_DEV - 1, SUB)),
            pltpu.SemaphoreType.DMA((2, N_DEV - 1, SUB)),
        ],
        compiler_params=pltpu.CompilerParams(
            collective_id=0,
            vmem_limit_bytes=60 * 1024 * 1024,
        ),
    )(mids, ids.reshape(T, 1), E)


# device time: 182503 ns/iter; 1.0562x vs baseline; 1.0009x over previous
import jax
import jax.numpy as jnp
from jax import lax
from jax.experimental import pallas as pl
from jax.experimental.pallas import tpu as pltpu

N_DEV = 4
T = 4096
D = 2048
HD = D // 2
V_SHARD = 8192
C = T // N_DEV
SUB = 2
S = C // SUB

_GROUP_OFFSET = (0, -1, 1, 2)


def kernel(ids, E):
    mids = jnp.bitwise_and(ids, V_SHARD - 1)

    def body(mids_smem, ids_v, e_hbm, out_hbm, g_ref, rs_recv, accbuf,
             gsem, outsem, snd_sems, rs_recv_sems, ag_recv_sems):
        my = lax.axis_index("i")
        left = lax.rem(my + N_DEV - 1, N_DEV)
        right = lax.rem(my + 1, N_DEV)
        base = my * V_SHARD

        def group_chunk(o):
            return lax.rem(my + _GROUP_OFFSET[o] + N_DEV, N_DEV)

        def issue_sub(o, j):
            t0 = group_chunk(o) * C + j * S

            def st(i, carry):
                l = mids_smem[t0 + i]
                pltpu.make_async_copy(
                    e_hbm.at[pl.ds(l, 1), :],
                    g_ref.at[pl.ds(t0 + i, 1), :],
                    gsem.at[o, j],
                ).start()
                return carry

            lax.fori_loop(0, S, st, 0, unroll=8)

        def drain_sub(o, j):
            pltpu.make_async_copy(
                e_hbm.at[pl.ds(0, S), :],
                g_ref.at[pl.ds(group_chunk(o) * C, S), :],
                gsem.at[o, j],
            ).wait()

        def xval(c, d, j):
            idv = ids_v[pl.ds(c * C + j * S, S), :]
            ok = (idv >= base) & (idv < base + V_SHARD)
            gg = g_ref[pl.ds(c * C + j * S, S), pl.ds(d * HD, HD)]
            return jnp.where(ok, gg, 0.0).astype(jnp.bfloat16)

        dests = (right, left)

        def rchunk(d, s):
            step = (s + 1) if d else -(s + 1)
            return lax.rem(my + step + 2 * N_DEV, N_DEV)

        def rs_send(s, d, j):
            if s == 0:
                src = accbuf.at[d, pl.ds(j * S, S), :]
            else:
                src = rs_recv.at[d, s - 1, pl.ds(j * S, S), :]
            return pltpu.make_async_remote_copy(
                src_ref=src,
                dst_ref=rs_recv.at[d, s, pl.ds(j * S, S), :],
                send_sem=snd_sems.at[d, s, j],
                recv_sem=rs_recv_sems.at[d, s, j],
                device_id=(dests[d],),
                device_id_type=pl.DeviceIdType.MESH,
            )

        def ag_sl(h, d, j):
            gc = lax.rem(my + (h - 1 if d else 1 - h) + 2 * N_DEV, N_DEV)
            return out_hbm.at[pl.ds(gc * C + j * S, S), pl.ds(d * HD, HD)]

        def ag_send(h, d, j):
            sl = ag_sl(h, d, j)
            return pltpu.make_async_remote_copy(
                src_ref=sl,
                dst_ref=sl,
                send_sem=snd_sems.at[d, h, j],
                recv_sem=ag_recv_sems.at[d, h, j],
                device_id=(dests[d],),
                device_id_type=pl.DeviceIdType.MESH,
            )

        issue_sub(0, 0)
        issue_sub(0, 1)

        barrier_sem = pltpu.get_barrier_semaphore()
        for nbr in [left, right]:
            pl.semaphore_signal(
                barrier_sem, inc=1,
                device_id=(nbr,), device_id_type=pl.DeviceIdType.MESH,
            )
        pl.semaphore_wait(barrier_sem, 2)

        rs_rdmas = []
        for j in range(SUB):
            drain_sub(0, j)
            for d in (0, 1):
                accbuf[d, pl.ds(j * S, S), :] = xval(my, d, j)
                r = rs_send(0, d, j)
                r.start()
                rs_rdmas.append(r)
        issue_sub(1, 0)
        issue_sub(2, 0)
        issue_sub(1, 1)
        issue_sub(2, 1)
        for s in range(N_DEV - 1):
            if s == 1:
                issue_sub(3, 0)
                issue_sub(3, 1)
            for j in range(SUB):
                if s == 0:
                    drain_sub(1, j)
                    drain_sub(2, j)
                elif s == 1:
                    drain_sub(3, j)
                for d in (0, 1):
                    rs_send(s, d, j).wait_recv()
                    rc = rchunk(d, s)
                    acc = (rs_recv[d, s, pl.ds(j * S, S), :]
                           + xval(rc, d, j))
                    if s < N_DEV - 2:
                        rs_recv[d, s, pl.ds(j * S, S), :] = acc
                        r = rs_send(s + 1, d, j)
                        r.start()
                        rs_rdmas.append(r)
                    else:
                        accbuf[d, pl.ds(j * S, S), :] = acc
                        pltpu.make_async_copy(
                            accbuf.at[d, pl.ds(j * S, S), :],
                            out_hbm.at[pl.ds(rc * C + j * S, S),
                                       pl.ds(d * HD, HD)],
                            outsem.at[d, j],
                        ).start()
        for r in rs_rdmas:
            r.wait_send()

        ag_rdmas = []
        for j in range(SUB):
            for d in (0, 1):
                pltpu.make_async_copy(
                    accbuf.at[d, pl.ds(j * S, S), :],
                    out_hbm.at[pl.ds(rchunk(d, N_DEV - 2) * C + j * S, S),
                               pl.ds(d * HD, HD)],
                    outsem.at[d, j],
                ).wait()
                r = ag_send(0, d, j)
                r.start()
                ag_rdmas.append(r)
        for h in range(N_DEV - 1):
            for j in range(SUB):
                for d in (0, 1):
                    ag_send(h, d, j).wait_recv()
                    if h < N_DEV - 2:
                        r = ag_send(h + 1, d, j)
                        r.start()
                        ag_rdmas.append(r)
        for r in ag_rdmas:
            r.wait_send()

    return pl.pallas_call(
        body,
        out_shape=jax.ShapeDtypeStruct((T, D), jnp.bfloat16),
        in_specs=[
            pl.BlockSpec(memory_space=pltpu.SMEM),
            pl.BlockSpec(memory_space=pltpu.VMEM),
            pl.BlockSpec(memory_space=pl.ANY),
        ],
        out_specs=pl.BlockSpec(memory_space=pl.ANY),
        scratch_shapes=[
            pltpu.VMEM((T, D), jnp.float32),
            pltpu.VMEM((2, N_DEV - 1, C, HD), jnp.bfloat16),
            pltpu.VMEM((2, C, HD), jnp.bfloat16),
            pltpu.SemaphoreType.DMA((N_DEV, SUB)),
            pltpu.SemaphoreType.DMA((2, SUB)),
            pltpu.SemaphoreType.DMA((2, N_DEV - 1, SUB)),
            pltpu.SemaphoreType.DMA((2, N_DEV - 1, SUB)),
            pltpu.SemaphoreType.DMA((2, N_DEV - 1, SUB)),
        ],
        compiler_params=pltpu.CompilerParams(
            collective_id=0,
            vmem_limit_bytes=60 * 1024 * 1024,
        ),
    )(mids, ids.reshape(T, 1), E)


# device time: 182054 ns/iter; 1.0588x vs baseline; 1.0025x over previous
import jax
import jax.numpy as jnp
from jax import lax
from jax.experimental import pallas as pl
from jax.experimental.pallas import tpu as pltpu

N_DEV = 4
T = 4096
D = 2048
HD = D // 2
V_SHARD = 8192
C = T // N_DEV
SUB = 2
S = C // SUB

_GROUP_OFFSET = (0, -1, 1, 2)


def kernel(ids, E):
    mids = jnp.bitwise_and(ids, V_SHARD - 1)

    def body(mids_smem, ids_v, e_hbm, out_hbm, g_ref, rs_recv, accbuf,
             xstage, gsem, outsem, snd_sems, rs_recv_sems, ag_recv_sems):
        my = lax.axis_index("i")
        left = lax.rem(my + N_DEV - 1, N_DEV)
        right = lax.rem(my + 1, N_DEV)
        base = my * V_SHARD

        def group_chunk(o):
            return lax.rem(my + _GROUP_OFFSET[o] + N_DEV, N_DEV)

        def issue_sub(o, j):
            t0 = group_chunk(o) * C + j * S

            def st(i, carry):
                l = mids_smem[t0 + i]
                pltpu.make_async_copy(
                    e_hbm.at[pl.ds(l, 1), :],
                    g_ref.at[pl.ds(t0 + i, 1), :],
                    gsem.at[o, j],
                ).start()
                return carry

            lax.fori_loop(0, S, st, 0, unroll=8)

        def drain_sub(o, j):
            pltpu.make_async_copy(
                e_hbm.at[pl.ds(0, S), :],
                g_ref.at[pl.ds(group_chunk(o) * C, S), :],
                gsem.at[o, j],
            ).wait()

        def xval(c, d, j):
            idv = ids_v[pl.ds(c * C + j * S, S), :]
            ok = (idv >= base) & (idv < base + V_SHARD)
            gg = g_ref[pl.ds(c * C + j * S, S), pl.ds(d * HD, HD)]
            return jnp.where(ok, gg, 0.0).astype(jnp.bfloat16)

        dests = (right, left)

        def rchunk(d, s):
            step = (s + 1) if d else -(s + 1)
            return lax.rem(my + step + 2 * N_DEV, N_DEV)

        def rs_send(s, d, j):
            if s == 0:
                src = accbuf.at[d, pl.ds(j * S, S), :]
            else:
                src = rs_recv.at[d, s - 1, pl.ds(j * S, S), :]
            return pltpu.make_async_remote_copy(
                src_ref=src,
                dst_ref=rs_recv.at[d, s, pl.ds(j * S, S), :],
                send_sem=snd_sems.at[d, s, j],
                recv_sem=rs_recv_sems.at[d, s, j],
                device_id=(dests[d],),
                device_id_type=pl.DeviceIdType.MESH,
            )

        def ag_sl(h, d, j):
            gc = lax.rem(my + (h - 1 if d else 1 - h) + 2 * N_DEV, N_DEV)
            return out_hbm.at[pl.ds(gc * C + j * S, S), pl.ds(d * HD, HD)]

        def ag_send(h, d, j):
            sl = ag_sl(h, d, j)
            return pltpu.make_async_remote_copy(
                src_ref=sl,
                dst_ref=sl,
                send_sem=snd_sems.at[d, h, j],
                recv_sem=ag_recv_sems.at[d, h, j],
                device_id=(dests[d],),
                device_id_type=pl.DeviceIdType.MESH,
            )

        issue_sub(0, 0)
        issue_sub(0, 1)

        barrier_sem = pltpu.get_barrier_semaphore()
        for nbr in [left, right]:
            pl.semaphore_signal(
                barrier_sem, inc=1,
                device_id=(nbr,), device_id_type=pl.DeviceIdType.MESH,
            )
        pl.semaphore_wait(barrier_sem, 2)

        rs_rdmas = []
        for j in range(SUB):
            drain_sub(0, j)
            for d in (0, 1):
                accbuf[d, pl.ds(j * S, S), :] = xval(my, d, j)
                r = rs_send(0, d, j)
                r.start()
                rs_rdmas.append(r)
        issue_sub(1, 0)
        issue_sub(2, 0)
        issue_sub(1, 1)
        issue_sub(2, 1)
        for s in range(N_DEV - 1):
            if s == 1:
                issue_sub(3, 0)
                issue_sub(3, 1)
            for j in range(SUB):
                if s == 0:
                    drain_sub(1, j)
                    drain_sub(2, j)
                elif s == 1:
                    drain_sub(3, j)
                for d in (0, 1):
                    xstage[d] = xval(rchunk(d, s), d, j)
                for d in (0, 1):
                    rs_send(s, d, j).wait_recv()
                    rc = rchunk(d, s)
                    acc = rs_recv[d, s, pl.ds(j * S, S), :] + xstage[d]
                    if s < N_DEV - 2:
                        rs_recv[d, s, pl.ds(j * S, S), :] = acc
                        r = rs_send(s + 1, d, j)
                        r.start()
                        rs_rdmas.append(r)
                    else:
                        accbuf[d, pl.ds(j * S, S), :] = acc
                        pltpu.make_async_copy(
                            accbuf.at[d, pl.ds(j * S, S), :],
                            out_hbm.at[pl.ds(rc * C + j * S, S),
                                       pl.ds(d * HD, HD)],
                            outsem.at[d, j],
                        ).start()
        for r in rs_rdmas:
            r.wait_send()

        ag_rdmas = []
        for j in range(SUB):
            for d in (0, 1):
                r = pltpu.make_async_remote_copy(
                    src_ref=accbuf.at[d, pl.ds(j * S, S), :],
                    dst_ref=ag_sl(0, d, j),
                    send_sem=snd_sems.at[d, 0, j],
                    recv_sem=ag_recv_sems.at[d, 0, j],
                    device_id=(dests[d],),
                    device_id_type=pl.DeviceIdType.MESH,
                )
                r.start()
                ag_rdmas.append(r)
        for h in range(N_DEV - 1):
            for j in range(SUB):
                for d in (0, 1):
                    ag_send(h, d, j).wait_recv()
                    if h < N_DEV - 2:
                        r = ag_send(h + 1, d, j)
                        r.start()
                        ag_rdmas.append(r)
        for r in ag_rdmas:
            r.wait_send()
        for j in range(SUB):
            for d in (0, 1):
                pltpu.make_async_copy(
                    accbuf.at[d, pl.ds(j * S, S), :],
                    out_hbm.at[pl.ds(rchunk(d, N_DEV - 2) * C + j * S, S),
                               pl.ds(d * HD, HD)],
                    outsem.at[d, j],
                ).wait()

    return pl.pallas_call(
        body,
        out_shape=jax.ShapeDtypeStruct((T, D), jnp.bfloat16),
        in_specs=[
            pl.BlockSpec(memory_space=pltpu.SMEM),
            pl.BlockSpec(memory_space=pltpu.VMEM),
            pl.BlockSpec(memory_space=pl.ANY),
        ],
        out_specs=pl.BlockSpec(memory_space=pl.ANY),
        scratch_shapes=[
            pltpu.VMEM((T, D), jnp.float32),
            pltpu.VMEM((2, N_DEV - 1, C, HD), jnp.bfloat16),
            pltpu.VMEM((2, C, HD), jnp.bfloat16),
            pltpu.VMEM((2, S, HD), jnp.bfloat16),
            pltpu.SemaphoreType.DMA((N_DEV, SUB)),
            pltpu.SemaphoreType.DMA((2, SUB)),
            pltpu.SemaphoreType.DMA((2, N_DEV - 1, SUB)),
            pltpu.SemaphoreType.DMA((2, N_DEV - 1, SUB)),
            pltpu.SemaphoreType.DMA((2, N_DEV - 1, SUB)),
        ],
        compiler_params=pltpu.CompilerParams(
            collective_id=0,
            vmem_limit_bytes=60 * 1024 * 1024,
        ),
    )(mids, ids.reshape(T, 1), E)
